# baseline (speedup 1.0000x reference)
import functools

import jax
import jax.numpy as jnp
from jax.experimental import pallas as pl

_F_TILE = 1024


def _moe_kernel(idx_ref, x_ref, gu_ref, dn_ref, out_ref):
    e = pl.program_id(0)
    fi = pl.program_id(1)

    mask = (idx_ref[...] == e).astype(jnp.float32)
    xm = x_ref[...] * mask

    gate_w = gu_ref[0, 0]
    up_w = gu_ref[0, 1]
    dims = (((1,), (1,)), ((), ()))
    gate = jax.lax.dot_general(xm, gate_w, dims, preferred_element_type=jnp.float32)
    up = jax.lax.dot_general(xm, up_w, dims, preferred_element_type=jnp.float32)
    hidden = gate * jax.nn.sigmoid(gate) * up

    down_w = dn_ref[0, :, pl.ds(fi * _F_TILE, _F_TILE)]
    partial = jax.lax.dot_general(
        hidden, down_w, dims, preferred_element_type=jnp.float32
    )

    @pl.when(jnp.logical_and(e == 0, fi == 0))
    def _init():
        out_ref[...] = partial

    @pl.when(jnp.logical_or(e != 0, fi != 0))
    def _acc():
        out_ref[...] += partial


@functools.partial(jax.jit, static_argnames=())
def kernel(x, expert_indices, gate_up_weight, down_weight):
    n_experts, two_dff, d_model = gate_up_weight.shape
    d_ff = two_dff // 2
    num_tokens = x.shape[0]

    nf = d_ff // _F_TILE

    gu = gate_up_weight.reshape(n_experts, 2, d_ff, d_model)
    idx = expert_indices.reshape(num_tokens, 1)

    out = pl.pallas_call(
        _moe_kernel,
        grid=(n_experts, nf),
        in_specs=[
            pl.BlockSpec((num_tokens, 1), lambda e, fi: (0, 0)),
            pl.BlockSpec((num_tokens, d_model), lambda e, fi: (0, 0)),
            pl.BlockSpec((1, 2, _F_TILE, d_model), lambda e, fi: (e, 0, fi, 0)),
            pl.BlockSpec((1, d_model, d_ff), lambda e, fi: (e, 0, 0)),
        ],
        out_specs=pl.BlockSpec((num_tokens, d_model), lambda e, fi: (0, 0)),
        out_shape=jax.ShapeDtypeStruct((num_tokens, d_model), jnp.float32),
    )(idx, x, gu, down_weight)
    return out


# speedup vs baseline: 1.0939x; 1.0939x over previous
import functools

import jax
import jax.numpy as jnp
from jax.experimental import pallas as pl

_F_TILE = 1024


def _moe_kernel(idx_ref, x_ref, gu_ref, dn_ref, out_ref):
    e = pl.program_id(0)
    fi = pl.program_id(1)

    mask = (idx_ref[...] == e).astype(jnp.float32)
    xm = x_ref[...] * mask

    gate_w = gu_ref[0, 0]
    up_w = gu_ref[0, 1]
    dims = (((1,), (1,)), ((), ()))
    gate = jax.lax.dot_general(xm, gate_w, dims, preferred_element_type=jnp.float32)
    up = jax.lax.dot_general(xm, up_w, dims, preferred_element_type=jnp.float32)
    hidden = gate * jax.nn.sigmoid(gate) * up

    down_w = dn_ref[0]
    partial = jax.lax.dot_general(
        hidden, down_w, dims, preferred_element_type=jnp.float32
    )

    @pl.when(jnp.logical_and(e == 0, fi == 0))
    def _init():
        out_ref[...] = partial

    @pl.when(jnp.logical_or(e != 0, fi != 0))
    def _acc():
        out_ref[...] += partial


@functools.partial(jax.jit, static_argnames=())
def kernel(x, expert_indices, gate_up_weight, down_weight):
    n_experts, two_dff, d_model = gate_up_weight.shape
    d_ff = two_dff // 2
    num_tokens = x.shape[0]

    nf = d_ff // _F_TILE

    gu = gate_up_weight.reshape(n_experts, 2, d_ff, d_model)
    idx = expert_indices.reshape(num_tokens, 1)

    out = pl.pallas_call(
        _moe_kernel,
        grid=(n_experts, nf),
        in_specs=[
            pl.BlockSpec((num_tokens, 1), lambda e, fi: (0, 0)),
            pl.BlockSpec((num_tokens, d_model), lambda e, fi: (0, 0)),
            pl.BlockSpec((1, 2, _F_TILE, d_model), lambda e, fi: (e, 0, fi, 0)),
            pl.BlockSpec((1, d_model, _F_TILE), lambda e, fi: (e, 0, fi)),
        ],
        out_specs=pl.BlockSpec((num_tokens, d_model), lambda e, fi: (0, 0)),
        out_shape=jax.ShapeDtypeStruct((num_tokens, d_model), jnp.float32),
    )(idx, x, gu, down_weight)
    return out
